# baseline (speedup 1.0000x reference)
import functools

import jax
import jax.numpy as jnp
from jax import lax
from jax.experimental import pallas as pl
from jax.experimental.pallas import tpu as pltpu
from jax.experimental.pallas import tpu_sc as plsc

GROUP = 1024
NUM_WORKERS = 32
LANES = 16


def _pick_block_rows(rows):
    best = 128
    br = 128
    while br <= 3200:
        if rows % br == 0:
            best = br
        br += 128
    return best


def _mask_kernel_body(groups_per_blk, u_ref, ae_ref, x_ref, wd_ref, exp_ref,
                      b_ref, na_ref, me_ref, pref_ref, carry):
    blk = pl.program_id(0)
    x = x_ref[...]
    logits = jnp.dot(x, wd_ref[...], preferred_element_type=jnp.float32)
    logits = logits + b_ref[0, 0]
    ae = ae_ref[...]
    probs = jax.nn.sigmoid(logits) * ae
    degens = (u_ref[...] < probs).astype(jnp.float32)
    na = ae * (1.0 - degens)
    na_ref[...] = na
    keep = (na > 0.0).astype(jnp.int32)
    m128 = jnp.dot(keep.astype(jnp.float32), exp_ref[...],
                   preferred_element_type=jnp.float32)
    me_ref[...] = jnp.where(m128 > 0.0, x, 0.0)

    @pl.when(blk == 0)
    def _():
        carry[0] = 0
        pref_ref[0] = 0

    c = carry[0]
    rows_per_group = 128
    for g in range(groups_per_blk):
        cnt = jnp.sum(keep[g * rows_per_group:(g + 1) * rows_per_group, :])
        c = c + cnt
        pref_ref[blk * groups_per_blk + g + 1] = c
    carry[0] = c


def _compute_mask(edges, active_edges, W, b, u):
    E = edges.shape[0]
    rows = E // 8
    br = _pick_block_rows(rows)
    nblk = rows // br
    groups_per_blk = (br * 8) // GROUP
    nsb = E // GROUP
    npref = ((nsb + 1 + 15) // 16) * 16

    x = edges.reshape(rows, 128)
    u8 = u.reshape(rows, 8)
    ae8 = active_edges.reshape(rows, 8)
    wd = jnp.kron(jnp.eye(8, dtype=jnp.float32), W)
    exp = jnp.kron(jnp.eye(8, dtype=jnp.float32),
                   jnp.ones((1, 16), jnp.float32))
    b2 = b.reshape(1, 1)

    naedges8, masked8, prefix = pl.pallas_call(
        functools.partial(_mask_kernel_body, groups_per_blk),
        grid=(nblk,),
        in_specs=[
            pl.BlockSpec((br, 8), lambda i: (i, 0)),
            pl.BlockSpec((br, 8), lambda i: (i, 0)),
            pl.BlockSpec((br, 128), lambda i: (i, 0)),
            pl.BlockSpec((128, 8), lambda i: (0, 0)),
            pl.BlockSpec((8, 128), lambda i: (0, 0)),
            pl.BlockSpec(memory_space=pltpu.MemorySpace.SMEM),
        ],
        out_specs=[
            pl.BlockSpec((br, 8), lambda i: (i, 0)),
            pl.BlockSpec((br, 128), lambda i: (i, 0)),
            pl.BlockSpec(memory_space=pltpu.MemorySpace.SMEM),
        ],
        out_shape=[
            jax.ShapeDtypeStruct((rows, 8), jnp.float32),
            jax.ShapeDtypeStruct((rows, 128), jnp.float32),
            jax.ShapeDtypeStruct((npref,), jnp.int32),
        ],
        scratch_shapes=[pltpu.SMEM((1,), jnp.int32)],
    )(u8, ae8, x, wd, exp, b2)
    return naedges8.reshape(E), masked8.reshape(E, 16), prefix, nsb


def _vread(ref, idx):
    base = (idx // LANES) * LANES
    v = ref[pl.ds(base, LANES)]
    lane = lax.iota(jnp.int32, LANES)
    return jnp.sum(jnp.where(lane == (idx - base), v, 0))


def _compact_body(nsb, n_fill, edges_hbm, na_hbm, snd_hbm, rcv_hbm, pref_hbm,
                  ne_hbm, ns_hbm, nr_hbm, nas_hbm,
                  ev, nav, sv, rv, dst2d, prefv, sem):
    wid = lax.axis_index("s") * 2 + lax.axis_index("c")
    pltpu.sync_copy(pref_hbm, prefv)
    k_total = _vread(prefv, nsb)

    vregs = GROUP // LANES
    ngroups = GROUP // 128

    def sub_block(j, _):
        gsb = wid + NUM_WORKERS * j
        start = gsb * GROUP
        kbase = _vread(prefv, gsb)
        dbase = k_total + (start - kbase)

        pltpu.sync_copy(edges_hbm.at[pl.ds(start, GROUP)], ev)
        pltpu.sync_copy(na_hbm.at[pl.ds(start, GROUP)], nav)
        pltpu.sync_copy(snd_hbm.at[pl.ds(start, GROUP)], sv)
        pltpu.sync_copy(rcv_hbm.at[pl.ds(start, GROUP)], rv)

        lanes = lax.iota(jnp.int32, LANES)

        def vreg_step(i, ck):
            off = i * LANES
            na = nav[pl.ds(off, LANES)]
            keep = na > 0.0
            ki = keep.astype(jnp.int32)
            cs = plsc.cumsum(ki)
            ek = cs - ki
            ed = lanes - ek
            dk = kbase + ck + ek
            dd = dbase + (off - ck) + ed
            dst2d[i // 8, pl.ds(LANES * lax.rem(i, 8), LANES)] = (
                jnp.where(keep, dk, dd))
            s = sv[pl.ds(off, LANES)]
            sv[pl.ds(off, LANES)] = jnp.where(keep, s, n_fill)
            r = rv[pl.ds(off, LANES)]
            rv[pl.ds(off, LANES)] = jnp.where(keep, r, n_fill)
            return ck + jnp.sum(ki)

        ck = lax.fori_loop(0, vregs, vreg_step, jnp.int32(0))

        copies = []
        for g in range(ngroups):
            idx = dst2d.at[g]
            copies.append(pltpu.async_copy(
                ev.at[pl.ds(128 * g, 128)], ne_hbm.at[idx], sem))
            copies.append(pltpu.async_copy(
                sv.at[pl.ds(128 * g, 128)], ns_hbm.at[idx], sem))
            copies.append(pltpu.async_copy(
                rv.at[pl.ds(128 * g, 128)], nr_hbm.at[idx], sem))
            copies.append(pltpu.async_copy(
                nav.at[pl.ds(128 * g, 128)], nas_hbm.at[idx], sem))
        for c in copies:
            c.wait()
        return 0

    nj = jnp.maximum(0, (nsb - 1 - wid) // NUM_WORKERS + 1)
    lax.fori_loop(0, nj, sub_block, 0)


def kernel(nodes, edges, receivers, senders, active_edges, active_nodes, W, b):
    E = edges.shape[0]
    n_fill = nodes.shape[0] - 1

    key_prob = jax.random.key(42)
    u = jax.random.uniform(key_prob, (E,))

    naedges, masked_edges, prefix, nsb = _compute_mask(
        edges, active_edges, W, b, u)

    mesh = plsc.VectorSubcoreMesh(core_axis_name="c", subcore_axis_name="s")
    npref = prefix.shape[0]
    new_edges, nsend, nrec, naedges_s = pl.kernel(
        functools.partial(_compact_body, nsb, n_fill),
        out_type=[
            jax.ShapeDtypeStruct((E, 16), jnp.float32),
            jax.ShapeDtypeStruct((E,), senders.dtype),
            jax.ShapeDtypeStruct((E,), receivers.dtype),
            jax.ShapeDtypeStruct((E,), jnp.float32),
        ],
        mesh=mesh,
        compiler_params=pltpu.CompilerParams(use_tc_tiling_on_sc=False,
                                             needs_layout_passes=False),
        scratch_types=[
            pltpu.VMEM((GROUP, 16), jnp.float32),
            pltpu.VMEM((GROUP,), jnp.float32),
            pltpu.VMEM((GROUP,), jnp.int32),
            pltpu.VMEM((GROUP,), jnp.int32),
            pltpu.VMEM((GROUP // 128, 128), jnp.int32),
            pltpu.VMEM((npref,), jnp.int32),
            pltpu.SemaphoreType.DMA,
        ],
    )(masked_edges, naedges, senders, receivers, prefix)

    return (new_edges, nsend, nrec, naedges_s)
